# Initial kernel scaffold; baseline (speedup 1.0000x reference)
import functools

import jax
import jax.numpy as jnp
from jax import lax
from jax.experimental import pallas as pl
from jax.experimental.pallas import tpu as pltpu
from jax.experimental.pallas import tpu_sc as plsc

N = 10000
NP = 10240
D = 128
XAW = 256
WPR = NP // 32
NC = 2
NS = 16
NW = NC * NS
HALF = NP // NC
RB_ROWS = HALF * (WPR // 16)

_MESH = plsc.VectorSubcoreMesh(core_axis_name="c", subcore_axis_name="s")


def _zero_shared(shared, zbuf, sid, rows_total, width):
  zr = zbuf.shape[0]
  def zrow(r, _):
    for k in range(width // 16):
      zbuf[r, pl.ds(k * 16, 16)] = jnp.zeros((16,), zbuf.dtype)
    return 0
  lax.fori_loop(0, zr, zrow, 0)
  per_tile = rows_total // NS
  nrep = per_tile // zr
  base = sid * per_tile
  def zcp(k, _):
    pltpu.sync_copy(zbuf, shared.at[pl.ds(base + k * zr, zr)])
    return 0
  lax.fori_loop(0, nrep, zcp, 0)


_K1_CH = 80

def _k1_body(ridx_h, lane_h, val_h, rb_h,
             ridxv, lanev, valv, stage, zbuf, accsh, sem):
  del sem
  c = lax.axis_index("c")
  sid = lax.axis_index("s")
  _zero_shared(accsh, zbuf, sid, RB_ROWS, 16)
  plsc.subcore_barrier()
  iot = lax.iota(jnp.int32, 16)
  e_total = ridx_h.shape[1]
  per_tile = e_total // NS
  nch = per_tile // _K1_CH
  base0 = sid * per_tile
  def chunk(i, _):
    b = base0 + i * _K1_CH
    pltpu.sync_copy(ridx_h.at[c, pl.ds(b, _K1_CH)], ridxv)
    pltpu.sync_copy(lane_h.at[c, pl.ds(b, _K1_CH)], lanev)
    pltpu.sync_copy(val_h.at[c, pl.ds(b, _K1_CH)], valv)
    for e in range(_K1_CH):
      stage[e, :] = jnp.where(iot == lanev[e], valv[e], 0)
    pltpu.sync_copy(stage, accsh.at[ridxv], add=True)
    return 0
  lax.fori_loop(0, nch, chunk, 0)
  plsc.subcore_barrier()
  rows_per_tile = RB_ROWS // NS
  pltpu.sync_copy(
      accsh.at[pl.ds(sid * rows_per_tile, rows_per_tile)],
      rb_h.at[pl.ds(c * RB_ROWS + sid * rows_per_tile, rows_per_tile)])


def _run_k1(ridx, lane, val):
  fn = functools.partial(
      pl.kernel,
      out_type=jax.ShapeDtypeStruct((NC * RB_ROWS, 16), jnp.int32),
      mesh=_MESH,
      scratch_types=[
          pltpu.VMEM((_K1_CH,), jnp.int32),
          pltpu.VMEM((_K1_CH,), jnp.int32),
          pltpu.VMEM((_K1_CH,), jnp.int32),
          pltpu.VMEM((_K1_CH, 16), jnp.int32),
          pltpu.VMEM((64, 16), jnp.int32),
          pltpu.VMEM_SHARED((RB_ROWS, 16), jnp.int32),
          pltpu.SemaphoreType.DMA,
      ],
  )(_k1_body)
  return fn(ridx, lane, val)


def _k2_start(rb_h, sp_h, dl_h, sidx, dlv, gbuf, sem, base):
  pltpu.sync_copy(sp_h.at[pl.ds(base, 16)], sidx)
  pltpu.sync_copy(dl_h.at[pl.ds(base, 16)], dlv)
  return pltpu.async_copy(rb_h.at[sidx], gbuf, sem)


def _k2_process(acc, gbuf, dlv):
  for e in range(16):
    dle = dlv[e]
    for w in range(WPR // 16):
      sl = pl.ds(w * 16, 16)
      acc[dle, sl] = acc[dle, sl] | gbuf[e, sl]


def _k2_body(rb_h, sp_h, dl_h, pb_h, r2_h,
             pbv, sidx0, dlv0, sidx1, dlv1, g0, g1, acc, sem0, sem1):
  c = lax.axis_index("c")
  sid = lax.axis_index("s")
  wid = c * NS + sid
  lo = wid * 320
  pltpu.sync_copy(pb_h, pbv)
  pltpu.sync_copy(rb_h.at[pl.ds(lo, 320)], acc)
  def diag(r, _):
    d = lo + r
    w = d >> 5
    acc[r, w] = acc[r, w] | (1 << (d & 31))
    return 0
  lax.fori_loop(0, 320, diag, 0)
  b0 = pbv[wid]
  b1 = pbv[wid + 1]
  nch = (b1 - b0) >> 4
  last = nch - 1
  _k2_start(rb_h, sp_h, dl_h, sidx0, dlv0, g0, sem0, b0).wait()
  def pair(i, _):
    j1 = jnp.minimum(2 * i + 1, last)
    cp1 = _k2_start(rb_h, sp_h, dl_h, sidx1, dlv1, g1, sem1, b0 + j1 * 16)
    _k2_process(acc, g0, dlv0)
    cp1.wait()
    j2 = jnp.minimum(2 * i + 2, last)
    cp0 = _k2_start(rb_h, sp_h, dl_h, sidx0, dlv0, g0, sem0, b0 + j2 * 16)
    _k2_process(acc, g1, dlv1)
    cp0.wait()
    return 0
  lax.fori_loop(0, (nch + 1) // 2, pair, 0)
  _k2_process(acc, g0, dlv0)
  pltpu.sync_copy(acc, r2_h.at[pl.ds(lo, 320)])


def _run_k2(rb2, sp, dl, pb):
  fn = functools.partial(
      pl.kernel,
      out_type=jax.ShapeDtypeStruct((NP, WPR), jnp.int32),
      mesh=_MESH,
      scratch_types=[
          pltpu.VMEM((40,), jnp.int32),
          pltpu.VMEM((16,), jnp.int32),
          pltpu.VMEM((16,), jnp.int32),
          pltpu.VMEM((16,), jnp.int32),
          pltpu.VMEM((16,), jnp.int32),
          pltpu.VMEM((16, WPR), jnp.int32),
          pltpu.VMEM((16, WPR), jnp.int32),
          pltpu.VMEM((320, WPR), jnp.int32),
          pltpu.SemaphoreType.DMA,
          pltpu.SemaphoreType.DMA,
      ],
  )(_k2_body)
  return fn(rb2, sp, dl, pb)


def _k3_gather(nx_h, se_h, de_h, sidx, didx, gs, gd, sem, base):
  pltpu.sync_copy(se_h.at[pl.ds(base, 16)], sidx)
  pltpu.sync_copy(de_h.at[pl.ds(base, 16)], didx)
  a = pltpu.async_copy(nx_h.at[sidx], gs, sem)
  b = pltpu.async_copy(nx_h.at[didx], gd, sem)
  return a, b


def _k3_process(gs, gd, dotb, pbuf, stage, p_h, accsh, sidx, iot, base):
  for e in range(16):
    acc = gs[e, pl.ds(0, 16)] * gd[e, pl.ds(0, 16)]
    for k in range(1, 8):
      sl = pl.ds(k * 16, 16)
      acc = acc + gs[e, sl] * gd[e, sl]
    dotb[e] = jnp.sum(acc)
  pbuf[:] = jnp.exp(dotb[...])
  pltpu.sync_copy(pbuf, p_h.at[pl.ds(base, 16)])
  for e in range(16):
    stage[e, :] = jnp.where(iot == 0, pbuf[e], 0.0)
  pltpu.sync_copy(stage, accsh.at[sidx], add=True)


def _k3_body(nx_h, se_h, de_h, p_h, sp_h,
             sidx0, didx0, sidx1, didx1, g0s, g0d, g1s, g1d,
             dotb, pbuf, stage, zbuf, accsh, sem0, sem1):
  c = lax.axis_index("c")
  sid = lax.axis_index("s")
  wid = c * NS + sid
  _zero_shared(accsh, zbuf, sid, NP, 16)
  plsc.subcore_barrier()
  iot = lax.iota(jnp.int32, 16)
  e_total = se_h.shape[0]
  per_tile = e_total // NW
  nch = per_tile // 16
  base0 = wid * per_tile
  a, b = _k3_gather(nx_h, se_h, de_h, sidx0, didx0, g0s, g0d, sem0, base0)
  a.wait()
  b.wait()
  def pair(i, _):
    j1 = jnp.minimum(2 * i + 1, nch - 1)
    a1, b1 = _k3_gather(nx_h, se_h, de_h, sidx1, didx1, g1s, g1d, sem1,
                        base0 + j1 * 16)
    _k3_process(g0s, g0d, dotb, pbuf, stage, p_h, accsh, sidx0, iot,
                base0 + (2 * i) * 16)
    a1.wait()
    b1.wait()
    j2 = jnp.minimum(2 * i + 2, nch - 1)
    a0, b0 = _k3_gather(nx_h, se_h, de_h, sidx0, didx0, g0s, g0d, sem0,
                        base0 + j2 * 16)
    _k3_process(g1s, g1d, dotb, pbuf, stage, p_h, accsh, sidx1, iot,
                base0 + j1 * 16)
    a0.wait()
    b0.wait()
    return 0
  lax.fori_loop(0, nch // 2, pair, 0)
  plsc.subcore_barrier()
  rows_per_tile = NP // NS
  pltpu.sync_copy(
      accsh.at[pl.ds(sid * rows_per_tile, rows_per_tile)],
      sp_h.at[c, pl.ds(sid * rows_per_tile, rows_per_tile)])


def _run_k3(nx, se, de):
  e = se.shape[0]
  fn = functools.partial(
      pl.kernel,
      out_type=(jax.ShapeDtypeStruct((e,), jnp.float32),
                jax.ShapeDtypeStruct((NC, NP, 16), jnp.float32)),
      mesh=_MESH,
      scratch_types=[
          pltpu.VMEM((16,), jnp.int32),
          pltpu.VMEM((16,), jnp.int32),
          pltpu.VMEM((16,), jnp.int32),
          pltpu.VMEM((16,), jnp.int32),
          pltpu.VMEM((16, D), jnp.float32),
          pltpu.VMEM((16, D), jnp.float32),
          pltpu.VMEM((16, D), jnp.float32),
          pltpu.VMEM((16, D), jnp.float32),
          pltpu.VMEM((16,), jnp.float32),
          pltpu.VMEM((16,), jnp.float32),
          pltpu.VMEM((16, 16), jnp.float32),
          pltpu.VMEM((64, 16), jnp.float32),
          pltpu.VMEM_SHARED((NP, 16), jnp.float32),
          pltpu.SemaphoreType.DMA,
          pltpu.SemaphoreType.DMA,
      ],
  )(_k3_body)
  return fn(nx, se, de)


def _k4_body(weighted, x_h, se_h, de_h, p_h, s2_h, out_h,
             sidx0, didx0, sidx1, didx1, g0, g1, sb, pbuf, wtb, stage,
             zbuf, accsh, sem0, sem1):
  c = lax.axis_index("c")
  sid = lax.axis_index("s")
  wid = c * NS + sid
  _zero_shared(accsh, zbuf, sid, NP, 144)
  plsc.subcore_barrier()
  iot = lax.iota(jnp.int32, 16)
  zer = jnp.zeros((16,), jnp.int32)
  e_total = se_h.shape[0]
  per_tile = e_total // NW
  nch = per_tile // 16
  base0 = wid * per_tile

  def start(sidx, didx, gbuf, sem, base):
    pltpu.sync_copy(se_h.at[pl.ds(base, 16)], sidx)
    pltpu.sync_copy(de_h.at[pl.ds(base, 16)], didx)
    return pltpu.async_copy(x_h.at[didx], gbuf, sem)

  def process(gbuf, sidx, base):
    if weighted:
      pltpu.sync_copy(p_h.at[pl.ds(base, 16)], pbuf)
      pltpu.async_copy(s2_h.at[sidx], sb, sem0).wait()
      svals = plsc.load_gather(sb, [iot, zer])
      wtb[:] = pbuf[...] / svals
    for e in range(16):
      if weighted:
        w = wtb[e]
        for k in range(8):
          sl = pl.ds(k * 16, 16)
          stage[e, sl] = gbuf[e, sl] * w
        stage[e, pl.ds(128, 16)] = jnp.where(iot == 0, w, 0.0)
      else:
        for k in range(8):
          sl = pl.ds(k * 16, 16)
          stage[e, sl] = gbuf[e, sl]
        stage[e, pl.ds(128, 16)] = jnp.where(iot == 0, 1.0, 0.0)
    pltpu.sync_copy(stage, accsh.at[sidx], add=True)

  cp = start(sidx0, didx0, g0, sem1, base0)
  cp.wait()
  def pair(i, _):
    j1 = jnp.minimum(2 * i + 1, nch - 1)
    cp1 = start(sidx1, didx1, g1, sem1, base0 + j1 * 16)
    process(g0, sidx0, base0 + (2 * i) * 16)
    cp1.wait()
    j2 = jnp.minimum(2 * i + 2, nch - 1)
    cp0 = start(sidx0, didx0, g0, sem1, base0 + j2 * 16)
    process(g1, sidx1, base0 + j1 * 16)
    cp0.wait()
    return 0
  lax.fori_loop(0, nch // 2, pair, 0)
  plsc.subcore_barrier()
  rows_per_tile = NP // NS
  pltpu.sync_copy(
      accsh.at[pl.ds(sid * rows_per_tile, rows_per_tile)],
      out_h.at[c, pl.ds(sid * rows_per_tile, rows_per_tile)])


def _run_k4(weighted, x_t, se, de, p, s2):
  fn = functools.partial(
      pl.kernel,
      out_type=jax.ShapeDtypeStruct((NC, NP, 144), jnp.float32),
      mesh=_MESH,
      scratch_types=[
          pltpu.VMEM((16,), jnp.int32),
          pltpu.VMEM((16,), jnp.int32),
          pltpu.VMEM((16,), jnp.int32),
          pltpu.VMEM((16,), jnp.int32),
          pltpu.VMEM((16, D), jnp.float32),
          pltpu.VMEM((16, D), jnp.float32),
          pltpu.VMEM((16, 16), jnp.float32),
          pltpu.VMEM((16,), jnp.float32),
          pltpu.VMEM((16,), jnp.float32),
          pltpu.VMEM((16, 144), jnp.float32),
          pltpu.VMEM((64, 144), jnp.float32),
          pltpu.VMEM_SHARED((NP, 144), jnp.float32),
          pltpu.SemaphoreType.DMA,
          pltpu.SemaphoreType.DMA,
      ],
  )(functools.partial(_k4_body, weighted))
  return fn(x_t, se, de, p, s2)


_K5_CH = 400

def _k5_body(h_h, se_h, de_h, out_h,
             sidx0, didx0, sidx1, didx1, g0, g1, zbuf, accsh, sem0, sem1):
  c = lax.axis_index("c")
  sid = lax.axis_index("s")
  wid = c * NS + sid
  _zero_shared(accsh, zbuf, sid, NP, D)
  plsc.subcore_barrier()
  e_total = se_h.shape[0]
  per_tile = e_total // NW
  nch = per_tile // _K5_CH
  base0 = wid * per_tile

  def start(sidx, didx, gbuf, sem, base):
    pltpu.sync_copy(se_h.at[pl.ds(base, _K5_CH)], sidx)
    pltpu.sync_copy(de_h.at[pl.ds(base, _K5_CH)], didx)
    return pltpu.async_copy(h_h.at[sidx], gbuf, sem)

  cp = start(sidx0, didx0, g0, sem0, base0)
  cp.wait()
  def pair(i, _):
    j1 = jnp.minimum(2 * i + 1, nch - 1)
    cp1 = start(sidx1, didx1, g1, sem1, base0 + j1 * _K5_CH)
    pltpu.sync_copy(g0, accsh.at[didx0], add=True)
    cp1.wait()
    j2 = jnp.minimum(2 * i + 2, nch - 1)
    cp0 = start(sidx0, didx0, g0, sem0, base0 + j2 * _K5_CH)
    pltpu.sync_copy(g1, accsh.at[didx1], add=True)
    cp0.wait()
    return 0
  lax.fori_loop(0, nch // 2, pair, 0)
  plsc.subcore_barrier()
  rows_per_tile = NP // NS
  pltpu.sync_copy(
      accsh.at[pl.ds(sid * rows_per_tile, rows_per_tile)],
      out_h.at[c, pl.ds(sid * rows_per_tile, rows_per_tile)])


def _run_k5(h, se, de):
  fn = functools.partial(
      pl.kernel,
      out_type=jax.ShapeDtypeStruct((NC, NP, D), jnp.float32),
      mesh=_MESH,
      scratch_types=[
          pltpu.VMEM((_K5_CH,), jnp.int32),
          pltpu.VMEM((_K5_CH,), jnp.int32),
          pltpu.VMEM((_K5_CH,), jnp.int32),
          pltpu.VMEM((_K5_CH,), jnp.int32),
          pltpu.VMEM((_K5_CH, D), jnp.float32),
          pltpu.VMEM((_K5_CH, D), jnp.float32),
          pltpu.VMEM((64, D), jnp.float32),
          pltpu.VMEM_SHARED((NP, D), jnp.float32),
          pltpu.SemaphoreType.DMA,
          pltpu.SemaphoreType.DMA,
      ],
  )(_k5_body)
  return fn(h, se, de)


_BLK = 256
_GRID = NP // _BLK


def _tc_pre_body(x_ref, wgt_ref, bg_ref, nx_ref, xa_ref, glob_ref):
  xb = x_ref[...]
  i = pl.program_id(0)
  rows = i * _BLK + lax.broadcasted_iota(jnp.int32, (_BLK, 1), 0)
  valid = (rows < N).astype(jnp.float32)
  nrm = jnp.sqrt(jnp.sum(xb * xb, axis=1, keepdims=True))
  nx_ref[...] = xb / jnp.maximum(nrm, 1e-12)
  xa_ref[...] = jnp.concatenate(
      [xb, valid, jnp.zeros((_BLK, XAW - D - 1), jnp.float32)], axis=1)
  glob_ref[...] = (
      jnp.dot(xb, wgt_ref[...], preferred_element_type=jnp.float32)
      + bg_ref[...])


def _run_tc_pre(xp, wg_t, bg):
  return pl.pallas_call(
      _tc_pre_body,
      grid=(_GRID,),
      in_specs=[
          pl.BlockSpec((_BLK, D), lambda i: (i, 0)),
          pl.BlockSpec((D, D), lambda i: (0, 0)),
          pl.BlockSpec((1, D), lambda i: (0, 0)),
      ],
      out_specs=[
          pl.BlockSpec((_BLK, D), lambda i: (i, 0)),
          pl.BlockSpec((_BLK, XAW), lambda i: (i, 0)),
          pl.BlockSpec((_BLK, D), lambda i: (i, 0)),
      ],
      out_shape=[
          jax.ShapeDtypeStruct((NP, D), jnp.float32),
          jax.ShapeDtypeStruct((NP, XAW), jnp.float32),
          jax.ShapeDtypeStruct((NP, D), jnp.float32),
      ],
  )(xp, wg_t, bg)


def _tc_ego_body(r2_ref, xa_ref, ego_ref, ebuf):
  iot = lax.broadcasted_iota(jnp.int32, (1, 32), 1)
  acc = jnp.zeros((_BLK, XAW), jnp.float32)
  for g in range(WPR // 8):
    for k in range(8):
      wcol = r2_ref[:, (g * 8 + k)][:, None]
      bits = ((wcol >> iot) & 1).astype(jnp.float32)
      ebuf[:, pl.ds(k * 32, 32)] = bits
    acc = acc + jnp.dot(ebuf[...], xa_ref[pl.ds(g * 256, 256), :],
                        preferred_element_type=jnp.float32)
  cnt = jnp.maximum(acc[:, D:D + 1], 1e-12)
  ego_ref[...] = acc[:, :D] / cnt


def _run_tc_ego(r2, xa):
  return pl.pallas_call(
      _tc_ego_body,
      grid=(_GRID,),
      in_specs=[
          pl.BlockSpec((_BLK, WPR), lambda i: (i, 0)),
          pl.BlockSpec((NP, XAW), lambda i: (0, 0)),
      ],
      out_specs=pl.BlockSpec((_BLK, D), lambda i: (i, 0)),
      out_shape=jax.ShapeDtypeStruct((NP, D), jnp.float32),
      scratch_shapes=[pltpu.VMEM((_BLK, 256), jnp.float32)],
  )(r2, xa)


def _tc_s_body(sp_ref, s2_ref):
  ssum = sp_ref[0] + sp_ref[1]
  s2_ref[...] = jnp.broadcast_to(ssum[:, 0:1], ssum.shape)


def _run_tc_s(s_parts):
  blk = 1024
  return pl.pallas_call(
      _tc_s_body,
      grid=(NP // blk,),
      in_specs=[pl.BlockSpec((NC, blk, 16), lambda i: (0, i, 0))],
      out_specs=pl.BlockSpec((blk, 16), lambda i: (i, 0)),
      out_shape=jax.ShapeDtypeStruct((NP, 16), jnp.float32),
  )(s_parts)


def _tc_mid_body(ap_ref, bp_ref, ego_ref, xp_ref,
                 wet_ref, be_ref, wct_ref, bc_ref, wkt_ref, bk_ref,
                 he_ref, hc_ref, hk_ref):
  a = ap_ref[0] + ap_ref[1]
  b = bp_ref[0] + bp_ref[1]
  xb = xp_ref[...]
  den = b[:, D:D + 1]
  has = den > 0
  cut = jnp.where(has, b[:, :D] / jnp.maximum(den, 1e-12), xb)
  cosf = jnp.where(has, a[:, :D] / jnp.maximum(a[:, D:D + 1], 1e-12), xb)
  ego = ego_ref[...]
  he_ref[...] = (
      jnp.dot(ego, wet_ref[...], preferred_element_type=jnp.float32)
      + be_ref[...])
  hc_ref[...] = (
      jnp.dot(cut, wct_ref[...], preferred_element_type=jnp.float32)
      + bc_ref[...])
  hk_ref[...] = (
      jnp.dot(cosf, wkt_ref[...], preferred_element_type=jnp.float32)
      + bk_ref[...])


def _run_tc_mid(a_parts, b_parts, ego, xp, wet, be, wct, bc, wkt, bk):
  wspec = pl.BlockSpec((D, D), lambda i: (0, 0))
  bspec = pl.BlockSpec((1, D), lambda i: (0, 0))
  return pl.pallas_call(
      _tc_mid_body,
      grid=(_GRID,),
      in_specs=[
          pl.BlockSpec((NC, _BLK, 144), lambda i: (0, i, 0)),
          pl.BlockSpec((NC, _BLK, 144), lambda i: (0, i, 0)),
          pl.BlockSpec((_BLK, D), lambda i: (i, 0)),
          pl.BlockSpec((_BLK, D), lambda i: (i, 0)),
          wspec, bspec, wspec, bspec, wspec, bspec,
      ],
      out_specs=[pl.BlockSpec((_BLK, D), lambda i: (i, 0))] * 3,
      out_shape=[jax.ShapeDtypeStruct((NP, D), jnp.float32)] * 3,
  )(a_parts, b_parts, ego, xp, wet, be, wct, bc, wkt, bk)


def _tc_final_body(pe_ref, pc_ref, pk_ref, glob_ref,
                   w1_ref, w2_ref, w3_ref, w4_ref, bf_ref, out_ref):
  e1 = jnp.maximum(pe_ref[0] + pe_ref[1], 0.0)
  e2 = jnp.maximum(pc_ref[0] + pc_ref[1], 0.0)
  e3 = jnp.maximum(pk_ref[0] + pk_ref[1], 0.0)
  g = glob_ref[...]
  lg = (jnp.dot(e1, w1_ref[...], preferred_element_type=jnp.float32)
        + jnp.dot(e2, w2_ref[...], preferred_element_type=jnp.float32)
        + jnp.dot(e3, w3_ref[...], preferred_element_type=jnp.float32)
        + jnp.dot(g, w4_ref[...], preferred_element_type=jnp.float32)
        + bf_ref[...])
  m = jnp.max(lg, axis=1, keepdims=True)
  sub = lg - m
  out_ref[...] = sub - jnp.log(jnp.sum(jnp.exp(sub), axis=1, keepdims=True))


def _run_tc_final(pe, pc, pk, glob, w1, w2, w3, w4, bf):
  pspec = pl.BlockSpec((NC, _BLK, D), lambda i: (0, i, 0))
  wspec = pl.BlockSpec((D, D), lambda i: (0, 0))
  return pl.pallas_call(
      _tc_final_body,
      grid=(_GRID,),
      in_specs=[
          pspec, pspec, pspec,
          pl.BlockSpec((_BLK, D), lambda i: (i, 0)),
          wspec, wspec, wspec, wspec,
          pl.BlockSpec((1, D), lambda i: (0, 0)),
      ],
      out_specs=pl.BlockSpec((_BLK, D), lambda i: (i, 0)),
      out_shape=jax.ShapeDtypeStruct((NP, D), jnp.float32),
  )(pe, pc, pk, glob, w1, w2, w3, w4, bf)


def kernel(x, edge_index, W_ego, b_ego, W_cut, b_cut, W_cos, b_cos,
           W_glob, b_glob, W_fc, b_fc):
  src = edge_index[0].astype(jnp.int32)
  dst = edge_index[1].astype(jnp.int32)
  e = src.shape[0]

  key = dst * 16384 + src
  skey = jnp.sort(key)
  d_s = skey >> 14
  s_s = skey & 16383
  dup = jnp.concatenate(
      [jnp.zeros((1,), bool), skey[1:] == skey[:-1]])
  bit = jnp.left_shift(jnp.int32(1), s_s & 31)
  val = jnp.where(dup, 0, bit).astype(jnp.int32)
  lanes = ((s_s >> 5) & 15).astype(jnp.int32)
  own0 = d_s < HALF
  lrow = d_s * 20 + (s_s >> 9)
  ridx = jnp.stack([jnp.where(own0, lrow, 0),
                    jnp.where(own0, 0, lrow - HALF * 20)]).astype(jnp.int32)
  vals = jnp.stack([jnp.where(own0, val, 0),
                    jnp.where(own0, 0, val)]).astype(jnp.int32)
  lns = jnp.stack([lanes, lanes])

  tile_of_edge = d_s // 320
  bnd = jnp.searchsorted(d_s, jnp.arange(33, dtype=jnp.int32) * 320
                         ).astype(jnp.int32)
  lens = bnd[1:] - bnd[:-1]
  plens = jnp.maximum(((lens + 15) // 16) * 16, 16)
  poff = jnp.concatenate(
      [jnp.zeros((1,), jnp.int32), jnp.cumsum(plens, dtype=jnp.int32)])
  padlen = e + NW * 16
  slot_tile = jnp.clip(
      jnp.searchsorted(poff, jnp.arange(padlen, dtype=jnp.int32),
                       side="right") - 1, 0, NW - 1).astype(jnp.int32)
  pos = poff[tile_of_edge] + (jnp.arange(e, dtype=jnp.int32)
                              - bnd[tile_of_edge])
  sp = (slot_tile * 320).at[pos].set(s_s.astype(jnp.int32))
  dl = jnp.zeros((padlen,), jnp.int32).at[pos].set(
      (d_s - tile_of_edge * 320).astype(jnp.int32))
  pb = jnp.concatenate([poff, jnp.zeros((7,), jnp.int32)])

  xp = jnp.zeros((NP, D), jnp.float32).at[:N].set(x)

  nx, xa, glob = _run_tc_pre(xp, W_glob.T, b_glob[None, :])

  rb = _run_k1(ridx, lns, vals)
  rb2 = rb.reshape(NP, WPR)
  r2 = _run_k2(rb2, sp, dl, pb)
  ego = _run_tc_ego(r2, xa)

  p, s_parts = _run_k3(nx, src, dst)
  s2 = _run_tc_s(s_parts)
  a_parts = _run_k4(True, xp, src, dst, p, s2)
  b_parts = _run_k4(False, xp, src, dst, p, s2)

  he, hc, hk = _run_tc_mid(a_parts, b_parts, ego, xp,
                           W_ego.T, b_ego[None, :], W_cut.T, b_cut[None, :],
                           W_cos.T, b_cos[None, :])

  pe = _run_k5(he, src, dst)
  pc = _run_k5(hc, src, dst)
  pk = _run_k5(hk, src, dst)

  wft = W_fc.T
  out = _run_tc_final(pe, pc, pk, glob,
                      wft[0:D], wft[D:2 * D], wft[2 * D:3 * D],
                      wft[3 * D:4 * D], b_fc[None, :])
  return out[:N]


# speedup vs baseline: 2.0805x; 2.0805x over previous
import functools

import jax
import jax.numpy as jnp
from jax import lax
from jax.experimental import pallas as pl
from jax.experimental.pallas import tpu as pltpu
from jax.experimental.pallas import tpu_sc as plsc

N = 10000
NP = 10240
D = 128
XAW = 256
WPR = NP // 32
SUBR = 3
NC = 2
NS = 16
NW = NC * NS
HALF = NP // NC
RB_ROWS = HALF * SUBR
IPT = WPR

def _mesh():
  return plsc.VectorSubcoreMesh(
      core_axis_name="c", subcore_axis_name="s", num_cores=NC,
      num_subcores=NS)

_GDN = lax.GatherDimensionNumbers(
    offset_dims=(), collapsed_slice_dims=(0,), start_index_map=(0,))


def _permute(v, idx):
  return lax.gather(v, idx[:, None], _GDN, (1,),
                    mode=lax.GatherScatterMode.PROMISE_IN_BOUNDS)


def _allsum(v):
  iot = lax.iota(jnp.int32, 16)
  for st in (8, 4, 2, 1):
    v = v + _permute(v, iot ^ st)
  return v


def _zero_shared(shared, zbuf, sid, rows_total, width):
  zr = zbuf.shape[0]
  zv = jnp.zeros((16,), zbuf.dtype)
  for r in range(zr):
    for k in range(width // 16):
      zbuf[r, pl.ds(k * 16, 16)] = zv
  per_tile = rows_total // NS
  nrep = per_tile // zr
  base = sid * per_tile
  def zcp(k, _):
    pltpu.sync_copy(zbuf, shared.at[pl.ds(base + k * zr, zr)])
    return 0
  lax.fori_loop(0, nrep, zcp, 0)


_K1_CH = 16

def _k1_body(ridx_h, lane_h, val_h, rb_h,
             ridxv, lanev, valv, stage, zbuf, accsh, sem):
  del sem
  c = lax.axis_index("c")
  sid = lax.axis_index("s")
  _zero_shared(accsh, zbuf, sid, RB_ROWS, 128)
  plsc.subcore_barrier()
  iot = lax.iota(jnp.int32, 16)
  e_total = ridx_h.shape[0] // NC
  per_tile = e_total // NS
  nch = per_tile // _K1_CH
  base0 = c * e_total + sid * per_tile
  def chunk(i, _):
    b = base0 + i * _K1_CH
    pltpu.sync_copy(ridx_h.at[pl.ds(b, _K1_CH)], ridxv)
    pltpu.sync_copy(lane_h.at[pl.ds(b, _K1_CH)], lanev)
    pltpu.sync_copy(val_h.at[pl.ds(b, _K1_CH)], valv)
    for g in range(_K1_CH // 16):
      lv = lanev[pl.ds(g * 16, 16)]
      vv = valv[pl.ds(g * 16, 16)]
      for j in range(16):
        lane = lv[j]
        grp = lane >> 4
        win = lane & 15
        for k in range(8):
          tgt = jnp.where(grp == k, win, -1)
          stage[g * 16 + j, pl.ds(k * 16, 16)] = jnp.where(
              iot == tgt, vv[j], 0)
    pltpu.sync_copy(stage, accsh.at[ridxv], add=True)
    return 0
  lax.fori_loop(0, nch, chunk, 0)
  plsc.subcore_barrier()
  rows_per_tile = RB_ROWS // NS
  pltpu.sync_copy(
      accsh.at[pl.ds(sid * rows_per_tile, rows_per_tile)],
      rb_h.at[pl.ds(c * RB_ROWS + sid * rows_per_tile, rows_per_tile)])


def _run_k1(ridx, lane, val):
  fn = functools.partial(
      pl.kernel,
      out_type=jax.ShapeDtypeStruct((NC * RB_ROWS, 128), jnp.int32),
      mesh=_mesh(),
      scratch_types=[
          pltpu.VMEM((_K1_CH,), jnp.int32),
          pltpu.VMEM((_K1_CH,), jnp.int32),
          pltpu.VMEM((_K1_CH,), jnp.int32),
          pltpu.VMEM((_K1_CH, 128), jnp.int32),
          pltpu.VMEM((8, 128), jnp.int32),
          pltpu.VMEM_SHARED((RB_ROWS, 128), jnp.int32),
          pltpu.SemaphoreType.DMA,
      ],
  )(_k1_body)
  return fn(ridx, lane, val)


_K2_CH = 8

def _k2_start(rb_h, sp_h, dl_h, sidx, dlv, gbuf, sem, base):
  base = pl.multiple_of(base, 8)
  pltpu.sync_copy(sp_h.at[pl.ds(base, _K2_CH)], sidx)
  pltpu.sync_copy(dl_h.at[pl.ds(base, _K2_CH)], dlv.at[pl.ds(0, _K2_CH)])
  return pltpu.async_copy(rb_h.at[sidx], gbuf, sem)


def _k2_process(acc, gbuf, dlv):
  dv = dlv[...]
  for e in range(_K2_CH):
    dle = dv[e]
    for w in range(WPR // 16):
      sl = pl.ds(w * 16, 16)
      acc[dle, sl] = acc[dle, sl] | gbuf[e, sl]


def _k2_body(rb_h, sp_h, dl_h, pb_h, r2_h,
             pbv, sidx0, dlv0, sidx1, dlv1, g0, g1, acc, sem0, sem1):
  c = lax.axis_index("c")
  sid = lax.axis_index("s")
  wid = c * NS + sid
  lo = wid * 320
  iot = lax.iota(jnp.int32, 16)
  pltpu.sync_copy(pb_h, pbv)
  zv = jnp.zeros((16,), jnp.int32)
  def zrow(r, _):
    for w in range(WPR // 16):
      acc[r, pl.ds(w * 16, 16)] = zv
    return 0
  lax.fori_loop(0, 320, zrow, 0)
  def diag(r, _):
    d = lo + r
    w = d >> 5
    base = (w >> 4) * 16
    sl = pl.ds(base, 16)
    acc[r, sl] = acc[r, sl] | jnp.where(iot == (w & 15), 1 << (d & 31), 0)
    return 0
  lax.fori_loop(0, 320, diag, 0)

  def pb_at(i):
    v = jnp.zeros((16,), jnp.int32)
    for g in range(3):
      grp = pbv[pl.ds(g * 16, 16)]
      v = v + jnp.where(iot + g * 16 == i, grp, 0)
    return _allsum(v)[0]

  b0 = pb_at(wid)
  b1 = pb_at(wid + 1)
  nch = (b1 - b0) >> 3
  last = nch - 1
  _k2_start(rb_h, sp_h, dl_h, sidx0, dlv0, g0, sem0, b0).wait()
  def pair(i, _):
    j1 = jnp.minimum(2 * i + 1, last)
    cp1 = _k2_start(rb_h, sp_h, dl_h, sidx1, dlv1, g1, sem1,
                    b0 + j1 * _K2_CH)
    _k2_process(acc, g0, dlv0)
    cp1.wait()
    j2 = jnp.minimum(2 * i + 2, last)
    cp0 = _k2_start(rb_h, sp_h, dl_h, sidx0, dlv0, g0, sem0,
                    b0 + j2 * _K2_CH)
    _k2_process(acc, g1, dlv1)
    cp0.wait()
    return 0
  lax.fori_loop(0, (nch + 1) // 2, pair, 0)
  _k2_process(acc, g0, dlv0)
  pltpu.sync_copy(acc, r2_h.at[pl.ds(lo, 320)])


def _run_k2(rb2, sp, dl, pb):
  fn = functools.partial(
      pl.kernel,
      out_type=jax.ShapeDtypeStruct((NP, WPR), jnp.int32),
      mesh=_mesh(),
      scratch_types=[
          pltpu.VMEM((48,), jnp.int32),
          pltpu.VMEM((_K2_CH,), jnp.int32),
          pltpu.VMEM((16,), jnp.int32),
          pltpu.VMEM((_K2_CH,), jnp.int32),
          pltpu.VMEM((16,), jnp.int32),
          pltpu.VMEM((_K2_CH, SUBR * 128), jnp.int32),
          pltpu.VMEM((_K2_CH, SUBR * 128), jnp.int32),
          pltpu.VMEM((320, WPR), jnp.int32),
          pltpu.SemaphoreType.DMA,
          pltpu.SemaphoreType.DMA,
      ],
  )(_k2_body)
  return fn(rb2, sp, dl, pb)


def _k3_gather(nx_h, se_h, de_h, sidx, didx, gs, gd, sem, base):
  pltpu.sync_copy(se_h.at[pl.ds(base, 16)], sidx)
  pltpu.sync_copy(de_h.at[pl.ds(base, 16)], didx)
  a = pltpu.async_copy(nx_h.at[sidx], gs, sem)
  b = pltpu.async_copy(nx_h.at[didx], gd, sem)
  return a, b


def _k3_process(gs, gd, pbuf, stage, p_h, accsh, sidx, iot, base):
  dots = jnp.zeros((16,), jnp.float32)
  for e in range(16):
    acc = gs[e, pl.ds(0, 16)] * gd[e, pl.ds(0, 16)]
    for k in range(1, 8):
      sl = pl.ds(k * 16, 16)
      acc = acc + gs[e, sl] * gd[e, sl]
    dots = jnp.where(iot == e, _allsum(acc)[0], dots)
  pv = jnp.exp(dots)
  pbuf[:] = pv
  pltpu.sync_copy(pbuf, p_h.at[pl.ds(base, 16)])
  zf = jnp.zeros((16,), jnp.float32)
  for e in range(16):
    stage[e, pl.ds(0, 16)] = jnp.where(
        iot == 0, pv[e], jnp.where(iot == 1, 1.0, 0.0))
    for k in range(1, 8):
      stage[e, pl.ds(k * 16, 16)] = zf
  pltpu.sync_copy(stage, accsh.at[sidx], add=True)


def _k3_body(nx_h, se_h, de_h, p_h, sp_h,
             sidx0, didx0, sidx1, didx1, g0s, g0d, g1s, g1d,
             pbuf, stage, zbuf, accsh, sem0, sem1):
  c = lax.axis_index("c")
  sid = lax.axis_index("s")
  wid = c * NS + sid
  _zero_shared(accsh, zbuf, sid, NP, 128)
  plsc.subcore_barrier()
  iot = lax.iota(jnp.int32, 16)
  e_total = se_h.shape[0]
  per_tile = e_total // NW
  nch = per_tile // 16
  base0 = wid * per_tile
  a, b = _k3_gather(nx_h, se_h, de_h, sidx0, didx0, g0s, g0d, sem0, base0)
  a.wait()
  b.wait()
  def pair(i, _):
    j1 = jnp.minimum(2 * i + 1, nch - 1)
    a1, b1 = _k3_gather(nx_h, se_h, de_h, sidx1, didx1, g1s, g1d, sem1,
                        base0 + j1 * 16)
    _k3_process(g0s, g0d, pbuf, stage, p_h, accsh, sidx0, iot,
                base0 + (2 * i) * 16)
    a1.wait()
    b1.wait()
    j2 = jnp.minimum(2 * i + 2, nch - 1)
    a0, b0 = _k3_gather(nx_h, se_h, de_h, sidx0, didx0, g0s, g0d, sem0,
                        base0 + j2 * 16)
    _k3_process(g1s, g1d, pbuf, stage, p_h, accsh, sidx1, iot,
                base0 + j1 * 16)
    a0.wait()
    b0.wait()
    return 0
  lax.fori_loop(0, nch // 2, pair, 0)
  plsc.subcore_barrier()
  rows_per_tile = NP // NS
  pltpu.sync_copy(
      accsh.at[pl.ds(sid * rows_per_tile, rows_per_tile)],
      sp_h.at[c, pl.ds(sid * rows_per_tile, rows_per_tile)])


def _run_k3(nx, se, de):
  e = se.shape[0]
  fn = functools.partial(
      pl.kernel,
      out_type=(jax.ShapeDtypeStruct((e,), jnp.float32),
                jax.ShapeDtypeStruct((NC, NP, 128), jnp.float32)),
      mesh=_mesh(),
      scratch_types=[
          pltpu.VMEM((16,), jnp.int32),
          pltpu.VMEM((16,), jnp.int32),
          pltpu.VMEM((16,), jnp.int32),
          pltpu.VMEM((16,), jnp.int32),
          pltpu.VMEM((16, D), jnp.float32),
          pltpu.VMEM((16, D), jnp.float32),
          pltpu.VMEM((16, D), jnp.float32),
          pltpu.VMEM((16, D), jnp.float32),
          pltpu.VMEM((16,), jnp.float32),
          pltpu.VMEM((16, 128), jnp.float32),
          pltpu.VMEM((16, 128), jnp.float32),
          pltpu.VMEM_SHARED((NP, 128), jnp.float32),
          pltpu.SemaphoreType.DMA,
          pltpu.SemaphoreType.DMA,
      ],
  )(_k3_body)
  return fn(nx, se, de)


def _k4_body(x_h, se_h, de_h, p_h, s2_h, out_h,
             sidx0, didx0, sidx1, didx1, g0, g1, sb, pbuf, stage,
             zbuf, accsh, sem0, sem1):
  c = lax.axis_index("c")
  sid = lax.axis_index("s")
  wid = c * NS + sid
  _zero_shared(accsh, zbuf, sid, NP, 128)
  plsc.subcore_barrier()
  iot = lax.iota(jnp.int32, 16)
  e_total = se_h.shape[0]
  per_tile = e_total // NW
  nch = per_tile // 16
  base0 = wid * per_tile

  def start(sidx, didx, gbuf, sem, base):
    pltpu.sync_copy(se_h.at[pl.ds(base, 16)], sidx)
    pltpu.sync_copy(de_h.at[pl.ds(base, 16)], didx)
    return pltpu.async_copy(x_h.at[didx], gbuf, sem)

  def process(gbuf, sidx, base):
    pltpu.sync_copy(p_h.at[pl.ds(base, 16)], pbuf)
    pltpu.async_copy(s2_h.at[sidx], sb, sem0).wait()
    svals = jnp.zeros((16,), jnp.float32)
    for e in range(16):
      svals = jnp.where(iot == e, sb[e, pl.ds(0, 16)][0], svals)
    wt16 = pbuf[...] / svals
    for e in range(16):
      w = wt16[e]
      for k in range(8):
        sl = pl.ds(k * 16, 16)
        stage[e, sl] = gbuf[e, sl] * w
    pltpu.sync_copy(stage, accsh.at[sidx], add=True)

  cp = start(sidx0, didx0, g0, sem1, base0)
  cp.wait()
  def pair(i, _):
    j1 = jnp.minimum(2 * i + 1, nch - 1)
    cp1 = start(sidx1, didx1, g1, sem1, base0 + j1 * 16)
    process(g0, sidx0, base0 + (2 * i) * 16)
    cp1.wait()
    j2 = jnp.minimum(2 * i + 2, nch - 1)
    cp0 = start(sidx0, didx0, g0, sem1, base0 + j2 * 16)
    process(g1, sidx1, base0 + j1 * 16)
    cp0.wait()
    return 0
  lax.fori_loop(0, nch // 2, pair, 0)
  plsc.subcore_barrier()
  rows_per_tile = NP // NS
  pltpu.sync_copy(
      accsh.at[pl.ds(sid * rows_per_tile, rows_per_tile)],
      out_h.at[c, pl.ds(sid * rows_per_tile, rows_per_tile)])


def _run_k4(x_t, se, de, p, s2):
  fn = functools.partial(
      pl.kernel,
      out_type=jax.ShapeDtypeStruct((NC, NP, 128), jnp.float32),
      mesh=_mesh(),
      scratch_types=[
          pltpu.VMEM((16,), jnp.int32),
          pltpu.VMEM((16,), jnp.int32),
          pltpu.VMEM((16,), jnp.int32),
          pltpu.VMEM((16,), jnp.int32),
          pltpu.VMEM((16, D), jnp.float32),
          pltpu.VMEM((16, D), jnp.float32),
          pltpu.VMEM((16, 128), jnp.float32),
          pltpu.VMEM((16,), jnp.float32),
          pltpu.VMEM((16, 128), jnp.float32),
          pltpu.VMEM((16, 128), jnp.float32),
          pltpu.VMEM_SHARED((NP, 128), jnp.float32),
          pltpu.SemaphoreType.DMA,
          pltpu.SemaphoreType.DMA,
      ],
  )(_k4_body)
  return fn(x_t, se, de, p, s2)


_K5_CH = 80

def _k5_body(h_h, ge_h, se_h, out_h,
             gidx0, sidx0, gidx1, sidx1, g0, g1, zbuf, accsh, sem0, sem1):
  c = lax.axis_index("c")
  sid = lax.axis_index("s")
  wid = c * NS + sid
  _zero_shared(accsh, zbuf, sid, NP, D)
  plsc.subcore_barrier()
  e_total = ge_h.shape[0]
  per_tile = e_total // NW
  nch = per_tile // _K5_CH
  base0 = wid * per_tile

  def start(gidx, sidx, gbuf, sem, base):
    pltpu.sync_copy(ge_h.at[pl.ds(base, _K5_CH)], gidx)
    pltpu.sync_copy(se_h.at[pl.ds(base, _K5_CH)], sidx)
    return pltpu.async_copy(h_h.at[gidx], gbuf, sem)

  cp = start(gidx0, sidx0, g0, sem0, base0)
  cp.wait()
  def pair(i, _):
    j1 = jnp.minimum(2 * i + 1, nch - 1)
    cp1 = start(gidx1, sidx1, g1, sem1, base0 + j1 * _K5_CH)
    pltpu.sync_copy(g0, accsh.at[sidx0], add=True)
    cp1.wait()
    j2 = jnp.minimum(2 * i + 2, nch - 1)
    cp0 = start(gidx0, sidx0, g0, sem0, base0 + j2 * _K5_CH)
    pltpu.sync_copy(g1, accsh.at[sidx1], add=True)
    cp0.wait()
    return 0
  lax.fori_loop(0, nch // 2, pair, 0)
  plsc.subcore_barrier()
  rows_per_tile = NP // NS
  pltpu.sync_copy(
      accsh.at[pl.ds(sid * rows_per_tile, rows_per_tile)],
      out_h.at[c, pl.ds(sid * rows_per_tile, rows_per_tile)])


def _run_k5(h, ge, se):
  fn = functools.partial(
      pl.kernel,
      out_type=jax.ShapeDtypeStruct((NC, NP, D), jnp.float32),
      mesh=_mesh(),
      scratch_types=[
          pltpu.VMEM((_K5_CH,), jnp.int32),
          pltpu.VMEM((_K5_CH,), jnp.int32),
          pltpu.VMEM((_K5_CH,), jnp.int32),
          pltpu.VMEM((_K5_CH,), jnp.int32),
          pltpu.VMEM((_K5_CH, D), jnp.float32),
          pltpu.VMEM((_K5_CH, D), jnp.float32),
          pltpu.VMEM((16, D), jnp.float32),
          pltpu.VMEM_SHARED((NP, D), jnp.float32),
          pltpu.SemaphoreType.DMA,
          pltpu.SemaphoreType.DMA,
      ],
  )(_k5_body)
  return fn(h, ge, se)


_BLK = 256
_GRID = NP // _BLK


def _tc_pre_body(x_ref, wgt_ref, bg_ref, nx_ref, xa_ref, glob_ref):
  xb = x_ref[...]
  i = pl.program_id(0)
  rows = i * _BLK + lax.broadcasted_iota(jnp.int32, (_BLK, 1), 0)
  valid = (rows < N).astype(jnp.float32)
  nrm = jnp.sqrt(jnp.sum(xb * xb, axis=1, keepdims=True))
  nx_ref[...] = xb / jnp.maximum(nrm, 1e-12)
  xa_ref[...] = jnp.concatenate(
      [xb, valid, jnp.zeros((_BLK, XAW - D - 1), jnp.float32)], axis=1)
  glob_ref[...] = (
      jnp.dot(xb, wgt_ref[...], preferred_element_type=jnp.float32)
      + bg_ref[...])


def _run_tc_pre(xp, wg_t, bg):
  return pl.pallas_call(
      _tc_pre_body,
      grid=(_GRID,),
      in_specs=[
          pl.BlockSpec((_BLK, D), lambda i: (i, 0)),
          pl.BlockSpec((D, D), lambda i: (0, 0)),
          pl.BlockSpec((1, D), lambda i: (0, 0)),
      ],
      out_specs=[
          pl.BlockSpec((_BLK, D), lambda i: (i, 0)),
          pl.BlockSpec((_BLK, XAW), lambda i: (i, 0)),
          pl.BlockSpec((_BLK, D), lambda i: (i, 0)),
      ],
      out_shape=[
          jax.ShapeDtypeStruct((NP, D), jnp.float32),
          jax.ShapeDtypeStruct((NP, XAW), jnp.float32),
          jax.ShapeDtypeStruct((NP, D), jnp.float32),
      ],
  )(xp, wg_t, bg)


def _tc_ego_body(r2_ref, xa_ref, ego_ref, ebuf):
  iot = lax.broadcasted_iota(jnp.int32, (1, 32), 1)
  acc = jnp.zeros((_BLK, XAW), jnp.float32)
  for g in range(WPR // 8):
    for k in range(8):
      wcol = r2_ref[:, (g * 8 + k):(g * 8 + k + 1)]
      bits = ((wcol >> iot) & 1).astype(jnp.float32)
      ebuf[:, pl.ds(k * 32, 32)] = bits
    acc = acc + jnp.dot(ebuf[...], xa_ref[pl.ds(g * 256, 256), :],
                        preferred_element_type=jnp.float32)
  cnt = jnp.maximum(acc[:, D:D + 1], 1e-12)
  ego_ref[...] = acc[:, :D] / cnt


def _run_tc_ego(r2, xa):
  return pl.pallas_call(
      _tc_ego_body,
      grid=(_GRID,),
      in_specs=[
          pl.BlockSpec((_BLK, WPR), lambda i: (i, 0)),
          pl.BlockSpec((NP, XAW), lambda i: (0, 0)),
      ],
      out_specs=pl.BlockSpec((_BLK, D), lambda i: (i, 0)),
      out_shape=jax.ShapeDtypeStruct((NP, D), jnp.float32),
      scratch_shapes=[pltpu.VMEM((_BLK, 256), jnp.float32)],
  )(r2, xa)


def _tc_s_body(sp_ref, s2_ref):
  ssum = sp_ref[0] + sp_ref[1]
  s2_ref[...] = jnp.broadcast_to(ssum[:, 0:1], ssum.shape)


def _run_tc_s(s_parts):
  blk = 1024
  return pl.pallas_call(
      _tc_s_body,
      grid=(NP // blk,),
      in_specs=[pl.BlockSpec((NC, blk, 128), lambda i: (0, i, 0))],
      out_specs=pl.BlockSpec((blk, 128), lambda i: (i, 0)),
      out_shape=jax.ShapeDtypeStruct((NP, 128), jnp.float32),
  )(s_parts)


def _tc_mid_body(ap_ref, bp_ref, sp_ref, ego_ref, xp_ref,
                 wet_ref, be_ref, wct_ref, bc_ref, wkt_ref, bk_ref,
                 he_ref, hc_ref, hk_ref):
  a = ap_ref[0] + ap_ref[1]
  b = bp_ref[0] + bp_ref[1]
  den = (sp_ref[0] + sp_ref[1])[:, 1:2]
  xb = xp_ref[...]
  hasf = (den > 0).astype(jnp.float32)
  cut = hasf * (b / jnp.maximum(den, 1e-12)) + (1.0 - hasf) * xb
  cosf = hasf * a + (1.0 - hasf) * xb
  ego = ego_ref[...]
  he_ref[...] = (
      jnp.dot(ego, wet_ref[...], preferred_element_type=jnp.float32)
      + be_ref[...])
  hc_ref[...] = (
      jnp.dot(cut, wct_ref[...], preferred_element_type=jnp.float32)
      + bc_ref[...])
  hk_ref[...] = (
      jnp.dot(cosf, wkt_ref[...], preferred_element_type=jnp.float32)
      + bk_ref[...])


def _run_tc_mid(a_parts, b_parts, s_parts, ego, xp, wet, be, wct, bc, wkt, bk):
  pspec = pl.BlockSpec((NC, _BLK, 128), lambda i: (0, i, 0))
  wspec = pl.BlockSpec((D, D), lambda i: (0, 0))
  bspec = pl.BlockSpec((1, D), lambda i: (0, 0))
  return pl.pallas_call(
      _tc_mid_body,
      grid=(_GRID,),
      in_specs=[
          pspec, pspec, pspec,
          pl.BlockSpec((_BLK, D), lambda i: (i, 0)),
          pl.BlockSpec((_BLK, D), lambda i: (i, 0)),
          wspec, bspec, wspec, bspec, wspec, bspec,
      ],
      out_specs=[pl.BlockSpec((_BLK, D), lambda i: (i, 0))] * 3,
      out_shape=[jax.ShapeDtypeStruct((NP, D), jnp.float32)] * 3,
  )(a_parts, b_parts, s_parts, ego, xp, wet, be, wct, bc, wkt, bk)


def _tc_final_body(pe_ref, pc_ref, pk_ref, glob_ref,
                   w1_ref, w2_ref, w3_ref, w4_ref, bf_ref, out_ref):
  e1 = jnp.maximum(pe_ref[0] + pe_ref[1], 0.0)
  e2 = jnp.maximum(pc_ref[0] + pc_ref[1], 0.0)
  e3 = jnp.maximum(pk_ref[0] + pk_ref[1], 0.0)
  g = glob_ref[...]
  lg = (jnp.dot(e1, w1_ref[...], preferred_element_type=jnp.float32)
        + jnp.dot(e2, w2_ref[...], preferred_element_type=jnp.float32)
        + jnp.dot(e3, w3_ref[...], preferred_element_type=jnp.float32)
        + jnp.dot(g, w4_ref[...], preferred_element_type=jnp.float32)
        + bf_ref[...])
  m = jnp.max(lg, axis=1, keepdims=True)
  sub = lg - m
  out_ref[...] = sub - jnp.log(jnp.sum(jnp.exp(sub), axis=1, keepdims=True))


def _run_tc_final(pe, pc, pk, glob, w1, w2, w3, w4, bf):
  pspec = pl.BlockSpec((NC, _BLK, D), lambda i: (0, i, 0))
  wspec = pl.BlockSpec((D, D), lambda i: (0, 0))
  return pl.pallas_call(
      _tc_final_body,
      grid=(_GRID,),
      in_specs=[
          pspec, pspec, pspec,
          pl.BlockSpec((_BLK, D), lambda i: (i, 0)),
          wspec, wspec, wspec, wspec,
          pl.BlockSpec((1, D), lambda i: (0, 0)),
      ],
      out_specs=pl.BlockSpec((_BLK, D), lambda i: (i, 0)),
      out_shape=jax.ShapeDtypeStruct((NP, D), jnp.float32),
  )(pe, pc, pk, glob, w1, w2, w3, w4, bf)


def kernel(x, edge_index, W_ego, b_ego, W_cut, b_cut, W_cos, b_cos,
           W_glob, b_glob, W_fc, b_fc):
  src = edge_index[0].astype(jnp.int32)
  dst = edge_index[1].astype(jnp.int32)
  e = src.shape[0]

  key = dst * 16384 + src
  skey = jnp.sort(key)
  d_s = skey >> 14
  s_s = skey & 16383
  dup = jnp.concatenate(
      [jnp.zeros((1,), bool), skey[1:] == skey[:-1]])
  bit = jnp.left_shift(jnp.int32(1), s_s & 31)
  val = jnp.where(dup, 0, bit).astype(jnp.int32)
  word = s_s >> 5
  lanes = (word & 127).astype(jnp.int32)
  own0 = d_s < HALF
  lrow = d_s * SUBR + (word >> 7)
  ridx = jnp.concatenate(
      [jnp.where(own0, lrow, 0),
       jnp.where(own0, 0, lrow - HALF * SUBR)]).astype(jnp.int32)
  vals = jnp.concatenate([jnp.where(own0, val, 0),
                          jnp.where(own0, 0, val)]).astype(jnp.int32)
  lns = jnp.concatenate([lanes, lanes])

  tile_of_edge = d_s // 320
  bnd = jnp.searchsorted(d_s, jnp.arange(33, dtype=jnp.int32) * 320
                         ).astype(jnp.int32)
  lens = bnd[1:] - bnd[:-1]
  plens = IPT + ((lens + 15) // 16) * 16
  poff = jnp.concatenate(
      [jnp.zeros((1,), jnp.int32), jnp.cumsum(plens, dtype=jnp.int32)])
  padlen = e + NW * (IPT + 16)
  sl_ar = jnp.arange(padlen, dtype=jnp.int32)
  slot_tile = jnp.clip(
      jnp.searchsorted(poff, sl_ar, side="right") - 1, 0, NW - 1
  ).astype(jnp.int32)
  off_in_tile = sl_ar - poff[slot_tile]
  ident = off_in_tile < IPT
  sp = jnp.where(ident, slot_tile * 320 + jnp.minimum(off_in_tile, 319),
                 slot_tile * 320)
  dl = jnp.where(ident, jnp.minimum(off_in_tile, 319), 0)
  pos = poff[tile_of_edge] + IPT + (jnp.arange(e, dtype=jnp.int32)
                                    - bnd[tile_of_edge])
  sp = sp.at[pos].set(s_s.astype(jnp.int32))
  dl = dl.at[pos].set((d_s - tile_of_edge * 320).astype(jnp.int32))
  pb = jnp.concatenate([poff, jnp.zeros((15,), jnp.int32)])

  xp = jnp.zeros((NP, D), jnp.float32).at[:N].set(x)

  nx, xa, glob = _run_tc_pre(xp, W_glob.T, b_glob[None, :])

  rb = _run_k1(ridx, lns, vals)
  rb2 = rb.reshape(NP, SUBR * 128)
  r2 = _run_k2(rb2, sp, dl, pb)
  ego = _run_tc_ego(r2, xa)

  p, s_parts = _run_k3(nx, src, dst)
  s2 = _run_tc_s(s_parts)
  a_parts = _run_k4(xp, src, dst, p, s2)
  b_parts = _run_k5(xp, dst, src)

  he, hc, hk = _run_tc_mid(a_parts, b_parts, s_parts, ego, xp,
                           W_ego.T, b_ego[None, :], W_cut.T, b_cut[None, :],
                           W_cos.T, b_cos[None, :])

  pe = _run_k5(he, src, dst)
  pc = _run_k5(hc, src, dst)
  pk = _run_k5(hk, src, dst)

  wft = W_fc.T
  out = _run_tc_final(pe, pc, pk, glob,
                      wft[0:D], wft[D:2 * D], wft[2 * D:3 * D],
                      wft[3 * D:4 * D], b_fc[None, :])
  return out[:N]
